# baseline (speedup 1.0000x reference)
import jax
import jax.numpy as jnp
from jax import lax
from jax.experimental import pallas as pl
from jax.experimental.pallas import tpu as pltpu
from jax.experimental.pallas import tpu_sc as plsc

BATCH = 4096
SEQ = 200
DIM = 64
VOCAB = 1000

B_SC = 2560
NC = 2
NS = 16
NW = NC * NS
BB = B_SC // NW
TB = 4
NT = BB // TB
NSL = DIM // 16
GSPLIT = ((0, 104), (104, 96))

B_TC = BATCH - B_SC
VPAD = 1024
TBM = 16
R = TBM * SEQ
TC_OFF = B_SC // TBM

MB = 32


def _sc_body(x_hbm, nucleo_hbm, pos_hbm, out_hbm,
             idx0, idx1, buf0, buf1, pos_v, table_sh,
             isem0, isem1, gsem0, gsem1, ssem0, ssem1):
    idxv = (idx0, idx1)
    buf = (buf0, buf1)
    isem = (isem0, isem1)
    gsem = (gsem0, gsem1)
    ssem = (ssem0, ssem1)
    wid = lax.axis_index("s") * NC + lax.axis_index("c")
    b0 = wid * BB

    @pl.when(lax.axis_index("s") == 0)
    def _stage_table():
        pltpu.sync_copy(nucleo_hbm, table_sh)

    plsc.subcore_barrier()
    pltpu.sync_copy(pos_hbm, pos_v)

    def stage_idx(i, s):
        pltpu.async_copy(
            x_hbm.at[pl.ds(b0 + i * TB, TB)], idxv[s], isem[s])

    def fire_gathers(s):
        for n in range(TB):
            for off, w in GSPLIT:
                pltpu.async_copy(
                    table_sh.at[idxv[s].at[n, pl.ds(off, w)]],
                    buf[s].at[n, pl.ds(off, w)], gsem[s])

    def drain_gathers(s):
        for n in range(TB):
            for off, w in GSPLIT:
                pltpu.make_async_copy(
                    table_sh.at[idxv[s].at[n, pl.ds(off, w)]],
                    buf[s].at[n, pl.ds(off, w)], gsem[s]).wait()

    def out_slice(i):
        return out_hbm.at[pl.ds(b0 + i * TB, TB)]

    stage_idx(0, 0)
    pltpu.make_async_copy(
        x_hbm.at[pl.ds(b0, TB)], idxv[0], isem[0]).wait()
    fire_gathers(0)

    def pair(g, carry):
        for b in range(2):
            i = g * 2 + b
            s, t = b, 1 - b

            @pl.when(i + 1 < NT)
            def _stage():
                stage_idx(i + 1, t)

                @pl.when(i >= 1)
                def _drain_store():
                    pltpu.make_async_copy(
                        buf[t], out_slice(i - 1), ssem[t]).wait()

                pltpu.make_async_copy(
                    x_hbm.at[pl.ds(b0 + (i + 1) * TB, TB)],
                    idxv[t], isem[t]).wait()
                fire_gathers(t)

            drain_gathers(s)

            def padd(p, carry2):
                posr = [pos_v[p, pl.ds(j * 16, 16)] for j in range(NSL)]
                for n in range(TB):
                    for j in range(NSL):
                        sl = pl.ds(j * 16, 16)
                        buf[s][n, p, sl] = buf[s][n, p, sl] + posr[j]
                return carry2

            lax.fori_loop(0, SEQ, padd, 0)
            pltpu.async_copy(buf[s], out_slice(i), ssem[s])
        return carry

    lax.fori_loop(0, NT // 2, pair, 0)

    pltpu.make_async_copy(buf[0], out_slice(NT - 2), ssem[0]).wait()
    pltpu.make_async_copy(buf[1], out_slice(NT - 1), ssem[1]).wait()


def _sc_part(X, nucleo_table, pos_table):
    mesh = plsc.VectorSubcoreMesh(core_axis_name="c", subcore_axis_name="s")
    k = pl.kernel(
        _sc_body,
        mesh=mesh,
        compiler_params=pltpu.CompilerParams(use_tc_tiling_on_sc=False),
        out_type=jax.ShapeDtypeStruct((B_SC, SEQ, DIM), jnp.float32),
        scratch_types=[
            pltpu.VMEM((TB, SEQ), jnp.int32),
            pltpu.VMEM((TB, SEQ), jnp.int32),
            pltpu.VMEM((TB, SEQ, DIM), jnp.float32),
            pltpu.VMEM((TB, SEQ, DIM), jnp.float32),
            pltpu.VMEM((SEQ, DIM), jnp.float32),
            pltpu.VMEM_SHARED((VOCAB, DIM), jnp.float32),
            pltpu.SemaphoreType.DMA,
            pltpu.SemaphoreType.DMA,
            pltpu.SemaphoreType.DMA,
            pltpu.SemaphoreType.DMA,
            pltpu.SemaphoreType.DMA,
            pltpu.SemaphoreType.DMA,
        ],
    )
    return k(X, nucleo_table, pos_table)


def _tc_body(x_ref, table_ref, pos_ref, out_ref):
    classes = jax.lax.broadcasted_iota(jnp.int32, (R, VPAD), 1)
    oh = (classes == x_ref[...]).astype(jnp.bfloat16)
    acc = jnp.dot(oh, table_ref[...], preferred_element_type=jnp.float32)
    out_ref[...] = acc + pos_ref[...]


def _tc_part(xf, nucleo_table, pos_table):
    table_bf = jnp.pad(nucleo_table, ((0, VPAD - VOCAB), (0, 0))).astype(
        jnp.bfloat16)
    pos_rep = jnp.tile(pos_table, (TBM, 1))
    return pl.pallas_call(
        _tc_body,
        grid=(B_TC // TBM,),
        in_specs=[
            pl.BlockSpec((R, 1), lambda i: (TC_OFF + i, 0)),
            pl.BlockSpec((VPAD, DIM), lambda i: (0, 0)),
            pl.BlockSpec((R, DIM), lambda i: (0, 0)),
        ],
        out_specs=pl.BlockSpec((R, DIM), lambda i: (TC_OFF + i, 0)),
        out_shape=jax.ShapeDtypeStruct((BATCH * SEQ, DIM), jnp.float32),
    )(xf, table_bf, pos_rep)


def _merge_body(sc_ref, full_ref, out_ref):
    out_ref[...] = sc_ref[...]


def _merge(sc_out, tc_full):
    sc_flat = sc_out.reshape(B_SC * SEQ, DIM)
    mrows = MB * SEQ
    return pl.pallas_call(
        _merge_body,
        grid=(B_SC // MB,),
        in_specs=[
            pl.BlockSpec((mrows, DIM), lambda i: (i, 0)),
            pl.BlockSpec((mrows, DIM), lambda i: (i, 0)),
        ],
        out_specs=pl.BlockSpec((mrows, DIM), lambda i: (i, 0)),
        out_shape=jax.ShapeDtypeStruct((BATCH * SEQ, DIM), jnp.float32),
        input_output_aliases={1: 0},
    )(sc_flat, tc_full)


def kernel(X, nucleo_table, pos_table):
    xf = X.reshape(BATCH * SEQ, 1)
    sc_out = _sc_part(X, nucleo_table, pos_table)
    tc_full = _tc_part(xf, nucleo_table, pos_table)
    out = _merge(sc_out, tc_full)
    return out.reshape(BATCH, SEQ, DIM)


# speedup vs baseline: 1.6086x; 1.6086x over previous
import jax
import jax.numpy as jnp
from jax import lax
from jax.experimental import pallas as pl
from jax.experimental.pallas import tpu as pltpu
from jax.experimental.pallas import tpu_sc as plsc

BATCH = 4096
SEQ = 200
DIM = 64
VOCAB = 1000
NC = 2
NS = 16
PG = 4
BG = 8
PP = SEQ // PG
BB = BATCH // BG
TB = 16
NT = BB // TB
NSL = DIM // 16


def _body(x_hbm, nucleo_hbm, pos_hbm, out_hbm,
          idx0, idx1, buf0, buf1, pos_v, table_sh,
          isem0, isem1, gsem0, gsem1, ssem0, ssem1):
    idxv = (idx0, idx1)
    buf = (buf0, buf1)
    isem = (isem0, isem1)
    gsem = (gsem0, gsem1)
    ssem = (ssem0, ssem1)
    wid = lax.axis_index("s") * NC + lax.axis_index("c")
    wp = wid % PG
    wb = wid // PG
    p0 = wp * PP
    b0 = wb * BB

    @pl.when(lax.axis_index("s") == 0)
    def _stage_table():
        pltpu.sync_copy(nucleo_hbm, table_sh)

    plsc.subcore_barrier()
    pltpu.sync_copy(pos_hbm.at[pl.ds(p0, PP)], pos_v)

    def stage_idx(i, s):
        pltpu.async_copy(
            x_hbm.at[wp, pl.ds(b0 + i * TB, TB)], idxv[s], isem[s])

    def fire_gathers(s):
        for n in range(TB):
            pltpu.async_copy(
                table_sh.at[idxv[s].at[n]], buf[s].at[n], gsem[s])

    def out_slice(i):
        return out_hbm.at[pl.ds(b0 + i * TB, TB), pl.ds(p0, PP)]

    stage_idx(0, 0)
    pltpu.make_async_copy(
        x_hbm.at[wp, pl.ds(b0, TB)], idxv[0], isem[0]).wait()
    fire_gathers(0)

    def pair(g, carry):
        for b in range(2):
            i = g * 2 + b
            s, t = b, 1 - b

            @pl.when(i + 1 < NT)
            def _stage():
                stage_idx(i + 1, t)

                @pl.when(i >= 1)
                def _drain_store():
                    pltpu.make_async_copy(
                        buf[t], out_slice(i - 1), ssem[t]).wait()

                pltpu.make_async_copy(
                    x_hbm.at[wp, pl.ds(b0 + (i + 1) * TB, TB)],
                    idxv[t], isem[t]).wait()
                fire_gathers(t)

            for n in range(TB):
                pltpu.make_async_copy(
                    table_sh.at[idxv[s].at[n]], buf[s].at[n],
                    gsem[s]).wait()

            def padd(p, carry2):
                posr = [pos_v[p, pl.ds(j * 16, 16)] for j in range(NSL)]
                for n in range(TB):
                    for j in range(NSL):
                        sl = pl.ds(j * 16, 16)
                        buf[s][n, p, sl] = buf[s][n, p, sl] + posr[j]
                return carry2

            lax.fori_loop(0, PP, padd, 0)
            pltpu.async_copy(buf[s], out_slice(i), ssem[s])
        return carry

    lax.fori_loop(0, NT // 2, pair, 0)

    pltpu.make_async_copy(buf[0], out_slice(NT - 2), ssem[0]).wait()
    pltpu.make_async_copy(buf[1], out_slice(NT - 1), ssem[1]).wait()


def kernel(X, nucleo_table, pos_table):
    xb = X.reshape(BATCH, PG, PP).transpose(1, 0, 2)
    mesh = plsc.VectorSubcoreMesh(core_axis_name="c", subcore_axis_name="s")
    k = pl.kernel(
        _body,
        mesh=mesh,
        compiler_params=pltpu.CompilerParams(use_tc_tiling_on_sc=False),
        out_type=jax.ShapeDtypeStruct((BATCH, SEQ, DIM), jnp.float32),
        scratch_types=[
            pltpu.VMEM((TB, PP), jnp.int32),
            pltpu.VMEM((TB, PP), jnp.int32),
            pltpu.VMEM((TB, PP, DIM), jnp.float32),
            pltpu.VMEM((TB, PP, DIM), jnp.float32),
            pltpu.VMEM((PP, DIM), jnp.float32),
            pltpu.VMEM_SHARED((VOCAB, DIM), jnp.float32),
            pltpu.SemaphoreType.DMA,
            pltpu.SemaphoreType.DMA,
            pltpu.SemaphoreType.DMA,
            pltpu.SemaphoreType.DMA,
            pltpu.SemaphoreType.DMA,
            pltpu.SemaphoreType.DMA,
        ],
    )
    return k(xb, nucleo_table, pos_table)
